# Initial kernel scaffold; baseline (speedup 1.0000x reference)
import jax
import jax.numpy as jnp
from jax.experimental import pallas as pl


def kernel(inputs, pretrain_table, id_table):
    raise NotImplementedError("write your pallas kernel here")


# speedup vs baseline: 1.0203x; 1.0203x over previous
import functools

import jax
import jax.numpy as jnp
from jax import lax
from jax.experimental import pallas as pl
from jax.experimental.pallas import tpu as pltpu
from jax.experimental.pallas import tpu_sc as plsc

VOCAB = 1000000
DIM = 64
BATCH = 4096
HIST = 50

_INFO = plsc.get_sparse_core_info()
NC = _INFO.num_cores
NS = _INFO.num_subcores
NW = NC * NS
B_TOTAL = BATCH * HIST
B_PER_W = B_TOTAL // NW
CHUNK = 128
NG = B_PER_W // CHUNK


def _emb_body(pt_hbm, it_hbm, idx_hbm, out_hbm, idx_v, rows_a, rows_b,
              sem_a, sem_b):
    wid = lax.axis_index("s") * NC + lax.axis_index("c")
    pltpu.sync_copy(idx_hbm.at[wid], idx_v)

    def g_body(g, carry):
        cp_a = pltpu.async_copy(pt_hbm.at[idx_v.at[g]], rows_a, sem_a)
        cp_b = pltpu.async_copy(it_hbm.at[idx_v.at[g]], rows_b, sem_b)
        cp_a.wait()
        cp_b.wait()

        def add_body(i, c2):
            for j in range(DIM // 16):
                s = pl.ds(j * 16, 16)
                plsc.addupdate(rows_a.at[i, s], rows_b[i, s])
            return c2

        lax.fori_loop(0, CHUNK, add_body, 0)
        pltpu.sync_copy(rows_a,
                        out_hbm.at[pl.ds(wid * B_PER_W + g * CHUNK, CHUNK)])
        return carry

    lax.fori_loop(0, NG, g_body, 0)


@jax.jit
def _emb(pretrain_table, id_table, idx):
    mesh = plsc.VectorSubcoreMesh(core_axis_name="c", subcore_axis_name="s")
    f = pl.kernel(
        _emb_body,
        out_type=jax.ShapeDtypeStruct((B_TOTAL, DIM), jnp.float32),
        mesh=mesh,
        scratch_types=[
            pltpu.VMEM((NG, CHUNK), jnp.int32),
            pltpu.VMEM((CHUNK, DIM), jnp.float32),
            pltpu.VMEM((CHUNK, DIM), jnp.float32),
            pltpu.SemaphoreType.DMA,
            pltpu.SemaphoreType.DMA,
        ],
        compiler_params=pltpu.CompilerParams(use_tc_tiling_on_sc=False),
    )
    return f(pretrain_table, id_table, idx)


def kernel(inputs, pretrain_table, id_table):
    idx = inputs.reshape(NW, NG, CHUNK)
    out = _emb(pretrain_table, id_table, idx)
    return out.reshape(BATCH, HIST, DIM)


# speedup vs baseline: 1.0633x; 1.0421x over previous
import jax
import jax.numpy as jnp
from jax import lax
from jax.experimental import pallas as pl
from jax.experimental.pallas import tpu as pltpu
from jax.experimental.pallas import tpu_sc as plsc

VOCAB = 1000000
DIM = 64
BATCH = 4096
HIST = 50

_INFO = plsc.get_sparse_core_info()
NC = _INFO.num_cores
NS = _INFO.num_subcores
NW = NC * NS
B_TOTAL = BATCH * HIST
B_PER_W = B_TOTAL // NW
CHUNK = 128
NG = B_PER_W // CHUNK
NBUF = 5


def _emb_body(pt_hbm, it_hbm, idx_hbm, out_hbm, idx_v, va, vb, vst,
              sem_a, sem_b, sem_st):
    wid = lax.axis_index("s") * NC + lax.axis_index("c")
    out_base = wid * B_PER_W
    pltpu.sync_copy(idx_hbm.at[wid], idx_v)

    def fire_gathers(g, b):
        pltpu.async_copy(pt_hbm.at[idx_v.at[g]], va.at[b], sem_a[b])
        pltpu.async_copy(it_hbm.at[idx_v.at[g]], vb.at[b], sem_b[b])

    def wait_gathers(g, b):
        pltpu.make_async_copy(pt_hbm.at[idx_v.at[g]], va.at[b],
                              sem_a[b]).wait()
        pltpu.make_async_copy(it_hbm.at[idx_v.at[g]], vb.at[b],
                              sem_b[b]).wait()

    def out_slice(g):
        return out_hbm.at[pl.ds(out_base + g * CHUNK, CHUNK)]

    def add_and_store(g, b):
        def add_body(i, c):
            for j in range(DIM // 16):
                s = pl.ds(j * 16, 16)
                vst[b, i, s] = va[b, i, s] + vb[b, i, s]
            return c

        lax.fori_loop(0, CHUNK, add_body, 0)
        pltpu.async_copy(vst.at[b], out_slice(g), sem_st[b])

    for b in range(NBUF):
        fire_gathers(b, b)

    for b in range(NBUF):
        wait_gathers(b, b)
        add_and_store(b, b)
        fire_gathers(b + NBUF, b)

    def outer_body(o, carry):
        for b in range(NBUF):
            g = o * NBUF + b
            wait_gathers(g, b)
            pltpu.make_async_copy(vst.at[b], out_slice(g - NBUF),
                                  sem_st[b]).wait()
            add_and_store(g, b)
            fire_gathers(g + NBUF, b)
        return carry

    lax.fori_loop(1, NG // NBUF - 1, outer_body, 0)

    for b in range(NBUF):
        g = NG - NBUF + b
        wait_gathers(g, b)
        pltpu.make_async_copy(vst.at[b], out_slice(g - NBUF),
                              sem_st[b]).wait()
        add_and_store(g, b)

    for b in range(NBUF):
        g = NG - NBUF + b
        pltpu.make_async_copy(vst.at[b], out_slice(g), sem_st[b]).wait()


@jax.jit
def _emb(pretrain_table, id_table, idx):
    mesh = plsc.VectorSubcoreMesh(core_axis_name="c", subcore_axis_name="s")
    f = pl.kernel(
        _emb_body,
        out_type=jax.ShapeDtypeStruct((B_TOTAL, DIM), jnp.float32),
        mesh=mesh,
        scratch_types=[
            pltpu.VMEM((NG, CHUNK), jnp.int32),
            pltpu.VMEM((NBUF, CHUNK, DIM), jnp.float32),
            pltpu.VMEM((NBUF, CHUNK, DIM), jnp.float32),
            pltpu.VMEM((NBUF, CHUNK, DIM), jnp.float32),
            [pltpu.SemaphoreType.DMA] * NBUF,
            [pltpu.SemaphoreType.DMA] * NBUF,
            [pltpu.SemaphoreType.DMA] * NBUF,
        ],
        compiler_params=pltpu.CompilerParams(use_tc_tiling_on_sc=False),
    )
    return f(pretrain_table, id_table, idx)


def kernel(inputs, pretrain_table, id_table):
    idx = inputs.reshape(NW, NG, CHUNK)
    out = _emb(pretrain_table, id_table, idx)
    return out.reshape(BATCH, HIST, DIM)
